# Initial kernel scaffold; baseline (speedup 1.0000x reference)
import jax
import jax.numpy as jnp
from jax.experimental import pallas as pl


def kernel(indices, table):
    raise NotImplementedError("write your pallas kernel here")


# speedup vs baseline: 54.9016x; 54.9016x over previous
import functools

import jax
import jax.numpy as jnp
from jax import lax
from jax.experimental import pallas as pl
from jax.experimental.pallas import tpu as pltpu
from jax.experimental.pallas import tpu_sc as plsc

_PRIMES = (179424941, 179425457)
_M = 200000
_D = 64

_NC, _NS, _L = 2, 16, 16
_NW = _NC * _NS

_P0 = _PRIMES[0] % _M
_P1 = _PRIMES[1] % _M
_C0 = (1024 * _PRIMES[0]) % _M
_C1 = (1024 * _PRIMES[1]) % _M

_CHUNK = 128


@functools.partial(jax.jit, static_argnums=(2,))
def _sc_lookup(flat_idx, table, n):
    per_w = n // _NW
    n_chunk = per_w // _CHUNK
    mesh = plsc.VectorSubcoreMesh(core_axis_name="c", subcore_axis_name="s")

    @functools.partial(
        pl.kernel,
        out_type=jax.ShapeDtypeStruct((n, _D), jnp.float32),
        mesh=mesh,
        compiler_params=pltpu.CompilerParams(use_tc_tiling_on_sc=False),
        scratch_types=[
            pltpu.VMEM((_CHUNK,), jnp.int32),
            pltpu.VMEM((_CHUNK,), jnp.int32),
            pltpu.VMEM((_CHUNK,), jnp.int32),
            pltpu.VMEM((_CHUNK, _D), jnp.float32),
            pltpu.VMEM((_CHUNK, _D), jnp.float32),
            pltpu.SemaphoreType.DMA,
        ],
    )
    def k(idx_hbm, table_hbm, out_hbm, idx_v, h0_v, h1_v, r0_v, r1_v, sem):
        wid = lax.axis_index("s") * jnp.int32(_NC) + lax.axis_index("c")
        base = wid * jnp.int32(per_w)

        @pl.loop(jnp.int32(0), jnp.int32(n_chunk))
        def _(g):
            off = base + g * jnp.int32(_CHUNK)
            pltpu.sync_copy(idx_hbm.at[pl.ds(off, _CHUNK)], idx_v)

            @pl.loop(jnp.int32(0), jnp.int32(_CHUNK), step=jnp.int32(_L))
            def _(j):
                v = idx_v[pl.ds(j, _L)]
                hi = lax.shift_right_logical(v, jnp.int32(10))
                lo = lax.bitwise_and(v, jnp.int32(1023))
                m = jnp.int32(_M)
                h0_v[pl.ds(j, _L)] = lax.rem(
                    hi * jnp.int32(_C0) + lo * jnp.int32(_P0), m)
                h1_v[pl.ds(j, _L)] = lax.rem(
                    hi * jnp.int32(_C1) + lo * jnp.int32(_P1), m)

            cp0 = pltpu.async_copy(table_hbm.at[h0_v], r0_v, sem)
            cp1 = pltpu.async_copy(table_hbm.at[h1_v], r1_v, sem)
            cp0.wait()
            cp1.wait()

            @pl.loop(jnp.int32(0), jnp.int32(_CHUNK))
            def _(i):
                for c in range(0, _D, _L):
                    r0_v[i, pl.ds(c, _L)] = (
                        r0_v[i, pl.ds(c, _L)] + r1_v[i, pl.ds(c, _L)]
                    )

            pltpu.sync_copy(r0_v, out_hbm.at[pl.ds(off, _CHUNK)])

    return k(flat_idx, table)


def kernel(indices, table):
    b, s = indices.shape
    flat = indices.reshape(-1).astype(jnp.int32)
    out = _sc_lookup(flat, table, flat.shape[0])
    return out.reshape(b, s, _D)


# speedup vs baseline: 62.6833x; 1.1417x over previous
import functools

import jax
import jax.numpy as jnp
from jax import lax
from jax.experimental import pallas as pl
from jax.experimental.pallas import tpu as pltpu
from jax.experimental.pallas import tpu_sc as plsc

_PRIMES = (179424941, 179425457)
_M = 200000
_D = 64

_NC, _NS, _L = 2, 16, 16
_NW = _NC * _NS

_P0 = _PRIMES[0] % _M
_P1 = _PRIMES[1] % _M
_C0 = (1024 * _PRIMES[0]) % _M
_C1 = (1024 * _PRIMES[1]) % _M

_GW = 128
_KG = 4
_CHUNK = _GW * _KG


@functools.partial(jax.jit, static_argnums=(2,))
def _sc_lookup(flat_idx, table, n):
    per_w = n // _NW
    n_chunk = per_w // _CHUNK
    mesh = plsc.VectorSubcoreMesh(core_axis_name="c", subcore_axis_name="s")

    @functools.partial(
        pl.kernel,
        out_type=jax.ShapeDtypeStruct((n, _D), jnp.float32),
        mesh=mesh,
        compiler_params=pltpu.CompilerParams(use_tc_tiling_on_sc=False),
        scratch_types=[
            pltpu.VMEM((_CHUNK,), jnp.int32),
            pltpu.VMEM((_KG, _GW), jnp.int32),
            pltpu.VMEM((_KG, _GW), jnp.int32),
            pltpu.VMEM((_CHUNK, _D), jnp.float32),
            pltpu.VMEM((_CHUNK, _D), jnp.float32),
            pltpu.SemaphoreType.DMA,
        ],
    )
    def k(idx_hbm, table_hbm, out_hbm, idx_v, h0_v, h1_v, r0_v, r1_v, sem):
        wid = lax.axis_index("s") * jnp.int32(_NC) + lax.axis_index("c")
        base = wid * jnp.int32(per_w)

        @pl.loop(jnp.int32(0), jnp.int32(n_chunk))
        def _(g):
            off = base + g * jnp.int32(_CHUNK)
            pltpu.sync_copy(idx_hbm.at[pl.ds(off, _CHUNK)], idx_v)

            for a in range(_KG):
                @pl.loop(jnp.int32(0), jnp.int32(_GW), step=jnp.int32(_L))
                def _(j, a=a):
                    v = idx_v[pl.ds(jnp.int32(a * _GW) + j, _L)]
                    hi = lax.shift_right_logical(v, jnp.int32(10))
                    lo = lax.bitwise_and(v, jnp.int32(1023))
                    m = jnp.int32(_M)
                    h0_v[a, pl.ds(j, _L)] = lax.rem(
                        hi * jnp.int32(_C0) + lo * jnp.int32(_P0), m)
                    h1_v[a, pl.ds(j, _L)] = lax.rem(
                        hi * jnp.int32(_C1) + lo * jnp.int32(_P1), m)

            copies = []
            for a in range(_KG):
                copies.append(pltpu.async_copy(
                    table_hbm.at[h0_v.at[jnp.int32(a)]],
                    r0_v.at[pl.ds(jnp.int32(a * _GW), _GW)], sem))
                copies.append(pltpu.async_copy(
                    table_hbm.at[h1_v.at[jnp.int32(a)]],
                    r1_v.at[pl.ds(jnp.int32(a * _GW), _GW)], sem))
            for cp in copies:
                cp.wait()

            @pl.loop(jnp.int32(0), jnp.int32(_CHUNK), step=jnp.int32(8))
            def _(i):
                for r in range(8):
                    for c in range(0, _D, _L):
                        row = i + jnp.int32(r)
                        r0_v[row, pl.ds(c, _L)] = (
                            r0_v[row, pl.ds(c, _L)] + r1_v[row, pl.ds(c, _L)]
                        )

            pltpu.sync_copy(r0_v, out_hbm.at[pl.ds(off, _CHUNK)])

    return k(flat_idx, table)


def kernel(indices, table):
    b, s = indices.shape
    flat = indices.reshape(-1).astype(jnp.int32)
    out = _sc_lookup(flat, table, flat.shape[0])
    return out.reshape(b, s, _D)
